# Initial kernel scaffold; baseline (speedup 1.0000x reference)
import jax
import jax.numpy as jnp
from jax.experimental import pallas as pl


def kernel(sentence_data, batch_sizes, gazetteers_data, word_table):
    raise NotImplementedError("write your pallas kernel here")


# speedup vs baseline: 2.5872x; 2.5872x over previous
import jax
import jax.numpy as jnp
from jax.experimental import pallas as pl
from jax.experimental.pallas import tpu as pltpu
from jax.experimental.pallas import tpu_sc as plsc

EMBED_DIM = 128
GAZ_DIM = 64
OUT_DIM = EMBED_DIM + GAZ_DIM
WINDOW = 128


def _embed_concat(sentence_data, gazetteers_data, word_table):
    num_tokens = sentence_data.shape[0]
    idx2d = sentence_data.reshape(1, num_tokens)
    mesh = plsc.VectorSubcoreMesh(core_axis_name="core",
                                  subcore_axis_name="subcore")

    @pl.kernel(
        out_type=jax.ShapeDtypeStruct((num_tokens, OUT_DIM), jnp.float32),
        mesh=mesh,
    )
    def kern(idx_hbm, gaz_hbm, table_hbm, out_hbm):
        def body(indices, i_vmem, o_vmem):
            (i,) = indices
            pltpu.sync_copy(table_hbm.at[i_vmem.at[0]],
                            o_vmem.at[:, pl.ds(0, EMBED_DIM)])
            pltpu.sync_copy(gaz_hbm.at[pl.ds(i * WINDOW, WINDOW)],
                            o_vmem.at[:, pl.ds(EMBED_DIM, GAZ_DIM)])

        pltpu.emit_pipeline(
            body,
            grid=(num_tokens // WINDOW,),
            in_specs=[
                pl.BlockSpec((1, WINDOW), lambda i: (0, i)),
            ],
            out_specs=[
                pl.BlockSpec((WINDOW, OUT_DIM), lambda i: (i, 0)),
            ],
            core_axis_name=("core", "subcore"),
            dimension_semantics=(pltpu.PARALLEL,),
            _explicit_indices=True,
        )(idx_hbm, out_hbm)

    return kern(idx2d, gazetteers_data, word_table)


def kernel(sentence_data, batch_sizes, gazetteers_data, word_table):
    out = _embed_concat(sentence_data, gazetteers_data, word_table)
    return out, batch_sizes


# speedup vs baseline: 2.7651x; 1.0688x over previous
import jax
import jax.numpy as jnp
from jax.experimental import pallas as pl
from jax.experimental.pallas import tpu as pltpu
from jax.experimental.pallas import tpu_sc as plsc

EMBED_DIM = 128
GAZ_DIM = 64
OUT_DIM = EMBED_DIM + GAZ_DIM
SUB = 128
GATHERS = 1
WINDOW = SUB * GATHERS


def _embed_concat(sentence_data, gazetteers_data, word_table):
    num_tokens = sentence_data.shape[0]
    idx2d = sentence_data.reshape(num_tokens // SUB, SUB)
    mesh = plsc.VectorSubcoreMesh(core_axis_name="core",
                                  subcore_axis_name="subcore")

    @pl.kernel(
        out_type=jax.ShapeDtypeStruct((num_tokens, OUT_DIM), jnp.float32),
        mesh=mesh,
        scratch_types=[pltpu.SemaphoreType.DMA,
                       pltpu.SemaphoreType.DMA],
    )
    def kern(idx_hbm, gaz_hbm, table_hbm, out_hbm, gsem, zsem):
        def body(indices, i_vmem, o_vmem):
            (i,) = indices
            zcp = pltpu.async_copy(
                gaz_hbm.at[pl.ds(i * WINDOW, WINDOW)],
                o_vmem.at[:, pl.ds(EMBED_DIM, GAZ_DIM)], zsem)
            cps = []
            for k in range(GATHERS):
                cps.append(pltpu.async_copy(
                    table_hbm.at[i_vmem.at[k]],
                    o_vmem.at[pl.ds(k * SUB, SUB), pl.ds(0, EMBED_DIM)],
                    gsem))
            for cp in cps:
                cp.wait()
            zcp.wait()

        pltpu.emit_pipeline(
            body,
            grid=(num_tokens // WINDOW,),
            in_specs=[
                pl.BlockSpec((GATHERS, SUB), lambda i: (i, 0)),
            ],
            out_specs=[
                pl.BlockSpec((WINDOW, OUT_DIM), lambda i: (i, 0)),
            ],
            core_axis_name=("core", "subcore"),
            dimension_semantics=(pltpu.PARALLEL,),
            _explicit_indices=True,
        )(idx_hbm, out_hbm)

    return kern(idx2d, gazetteers_data, word_table)


def kernel(sentence_data, batch_sizes, gazetteers_data, word_table):
    out = _embed_concat(sentence_data, gazetteers_data, word_table)
    return out, batch_sizes
